# baseline (speedup 1.0000x reference)
import functools

import jax
import jax.numpy as jnp
from jax import lax
from jax.experimental import pallas as pl
from jax.experimental.pallas import tpu as pltpu
from jax.experimental.pallas import tpu_sc as plsc

F_DIM = 64
HIST = 50
MU = 3.5
NORM = float(HIST) ** (-0.5)
NCHUNK = F_DIM // 16
NGRP = (HIST + 15) // 16 * 16


def _svdpp_body(user_hbm, item_hbm, imp_hbm, ub_hbm, ib_hbm, P_hbm, Q_hbm,
                Y_hbm, out_hbm, user_v, item_v, imp_v, bu_v, bi_v,
                pu_v, qi_v, rows_v, res_v, sem0, sem1, sem2, sem3):
    cid = lax.axis_index("c")
    sid = lax.axis_index("s")

    @pl.when(jnp.logical_and(cid == 0, sid == 0))
    def _():
        c0 = pltpu.async_copy(user_hbm, user_v.at[pl.ds(0, 1)], sem0)
        c1 = pltpu.async_copy(item_hbm, item_v.at[pl.ds(0, 1)], sem1)
        c2 = pltpu.async_copy(imp_hbm, imp_v.at[pl.ds(0, HIST)], sem2)
        c2.wait()
        impvecs = [imp_v[pl.ds(g * 16, 16)] for g in range(NGRP // 16)]
        c0.wait()
        c1.wait()
        u = user_v[...][0]
        it = item_v[...][0]
        ycopies = []
        for j in range(HIST):
            gj = pl.multiple_of((impvecs[j // 16][j % 16] >> 3) << 3, 8)
            ycopies.append(pltpu.async_copy(
                Y_hbm.at[pl.ds(gj, 8), :],
                rows_v.at[pl.ds(j * 8, 8), :], sem3))
        ug = pl.multiple_of((u >> 3) << 3, 8)
        ig = pl.multiple_of((it >> 3) << 3, 8)
        gP = pltpu.async_copy(P_hbm.at[pl.ds(ug, 8), :], pu_v, sem0)
        gQ = pltpu.async_copy(Q_hbm.at[pl.ds(ig, 8), :], qi_v, sem1)
        ub_base = pl.multiple_of((u >> 3) << 3, 8)
        ib_base = pl.multiple_of((it >> 3) << 3, 8)
        gb0 = pltpu.async_copy(ub_hbm.at[pl.ds(ub_base, 8)],
                               bu_v.at[pl.ds(0, 8)], sem2)
        gb1 = pltpu.async_copy(ib_hbm.at[pl.ds(ib_base, 8)],
                               bi_v.at[pl.ds(0, 8)], sem2)
        gP.wait()
        gQ.wait()
        gb0.wait()
        gb1.wait()
        for cp in ycopies:
            cp.wait()

        su = u & 7
        si = it & 7
        total = None
        for c in range(NCHUNK):
            sl = slice(c * 16, (c + 1) * 16)
            acc = None
            for j in range(HIST):
                sj = impvecs[j // 16][j % 16] & 7
                row = rows_v[j * 8 + sj, sl]
                acc = row if acc is None else acc + row
            p = pu_v[su, sl]
            q = qi_v[si, sl]
            t = p * (q + NORM * acc)
            total = t if total is None else total + t
        s = total[0]
        for i in range(1, 16):
            s = s + total[i]
        bu = plsc.load_gather(bu_v, [jnp.full((16,), su, jnp.int32)])[0]
        bi = plsc.load_gather(bi_v, [jnp.full((16,), si, jnp.int32)])[0]
        r = MU + bu + bi + s
        res_v[...] = jnp.full((16,), r, jnp.float32)
        pltpu.sync_copy(res_v.at[pl.ds(0, 1)], out_hbm)


_svdpp = functools.partial(
    pl.kernel,
    out_type=jax.ShapeDtypeStruct((1,), jnp.float32),
    mesh=plsc.VectorSubcoreMesh(core_axis_name="c", subcore_axis_name="s"),
    compiler_params=pltpu.CompilerParams(needs_layout_passes=False),
    scratch_types=[
        pltpu.VMEM((16,), jnp.int32),
        pltpu.VMEM((16,), jnp.int32),
        pltpu.VMEM((NGRP,), jnp.int32),
        pltpu.VMEM((16,), jnp.float32),
        pltpu.VMEM((16,), jnp.float32),
        pltpu.VMEM((8, F_DIM), jnp.float32),
        pltpu.VMEM((8, F_DIM), jnp.float32),
        pltpu.VMEM((HIST * 8, F_DIM), jnp.float32),
        pltpu.VMEM((16,), jnp.float32),
        pltpu.SemaphoreType.DMA,
        pltpu.SemaphoreType.DMA,
        pltpu.SemaphoreType.DMA,
        pltpu.SemaphoreType.DMA,
    ],
)(_svdpp_body)


def kernel(user, item, implicit_items, user_biases, item_biases, P, Q, Y):
    return _svdpp(
        user.astype(jnp.int32),
        item.astype(jnp.int32),
        implicit_items.astype(jnp.int32),
        user_biases.reshape(-1),
        item_biases.reshape(-1),
        P,
        Q,
        Y,
    )


# speedup vs baseline: 3.4958x; 3.4958x over previous
import functools

import jax
import jax.numpy as jnp
from jax import lax
from jax.experimental import pallas as pl
from jax.experimental.pallas import tpu as pltpu
from jax.experimental.pallas import tpu_sc as plsc

VOCAB = 100000
F_DIM = 64
HIST = 50
MU = 3.5
NORM = float(HIST) ** (-0.5)
NCHUNK = F_DIM // 16
NSLOT = 10
LAST_BASE = VOCAB - 128


def _block_base_lane(idx):
    return pl.multiple_of((idx >> 7) << 7, 128), idx & 127


def _svdpp_body(user_hbm, item_hbm, imp_hbm, ub_hbm, ib_hbm, P_hbm, Q_hbm,
                Y_hbm, out_hbm, user_v, item_v, imp_v, bu_v, bi_v, pu_v,
                qi_v, rows_v, res_v, sem0, sem1, sem2, sem3):
    cid = lax.axis_index("c")
    sid = lax.axis_index("s")

    @pl.when(jnp.logical_and(cid == 0, sid == 0))
    def _():
        iota = lax.iota(jnp.int32, 16)

        def col_chunks(ref3, slot, lane):
            slots = jnp.full((16,), slot, jnp.int32)
            lanes = jnp.full((16,), lane, jnp.int32)
            return [
                plsc.load_gather(ref3, [slots, iota + (c * 16), lanes])
                for c in range(NCHUNK)
            ]

        c0 = pltpu.async_copy(user_hbm, user_v.at[pl.ds(0, 1)], sem0)
        c1 = pltpu.async_copy(item_hbm, item_v.at[pl.ds(0, 1)], sem1)
        c2 = pltpu.async_copy(imp_hbm, imp_v.at[pl.ds(0, HIST)], sem2)
        c2.wait()
        impvecs = [imp_v[pl.ds(g * 16, 16)] for g in range((HIST + 15) // 16)]
        c0.wait()
        c1.wait()
        u = user_v[...][0]
        it = item_v[...][0]
        bu_base, u_lane = _block_base_lane(u)
        bi_base, i_lane = _block_base_lane(it)
        gP = pltpu.async_copy(P_hbm.at[:, pl.ds(bu_base, 128)],
                              pu_v.at[0], sem0)
        gQ = pltpu.async_copy(Q_hbm.at[:, pl.ds(bi_base, 128)],
                              qi_v.at[0], sem1)
        gb0 = pltpu.async_copy(ub_hbm.at[:, pl.ds(bu_base, 128)], bu_v, sem2)
        gb1 = pltpu.async_copy(ib_hbm.at[:, pl.ds(bi_base, 128)], bi_v, sem2)

        lanes = []
        copies = []

        def issue(j):
            base, lane = _block_base_lane(impvecs[j // 16][j % 16])
            lanes.append(lane)
            copies.append(pltpu.async_copy(
                Y_hbm.at[:, pl.ds(base, 128)],
                rows_v.at[j % NSLOT], sem3))

        for j in range(min(NSLOT, HIST)):
            issue(j)

        acc = [None] * NCHUNK
        for j in range(HIST):
            copies[j].wait()
            row = col_chunks(rows_v, j % NSLOT, lanes[j])
            for c in range(NCHUNK):
                acc[c] = row[c] if acc[c] is None else acc[c] + row[c]
            if j + NSLOT < HIST:
                issue(j + NSLOT)

        gP.wait()
        gQ.wait()
        gb0.wait()
        gb1.wait()
        pc = col_chunks(pu_v, 0, u_lane)
        qc = col_chunks(qi_v, 0, i_lane)
        total = None
        for c in range(NCHUNK):
            t = pc[c] * (qc[c] + NORM * acc[c])
            total = t if total is None else total + t
        s = total[0]
        for i in range(1, 16):
            s = s + total[i]
        zeros = jnp.zeros((16,), jnp.int32)
        bu = plsc.load_gather(
            bu_v, [zeros, jnp.full((16,), u_lane, jnp.int32)])[0]
        bi = plsc.load_gather(
            bi_v, [zeros, jnp.full((16,), i_lane, jnp.int32)])[0]
        r = MU + bu + bi + s
        res_v[...] = jnp.full((16,), r, jnp.float32)
        pltpu.sync_copy(res_v.at[pl.ds(0, 1)], out_hbm)


_svdpp = functools.partial(
    pl.kernel,
    out_type=jax.ShapeDtypeStruct((1,), jnp.float32),
    mesh=plsc.VectorSubcoreMesh(core_axis_name="c", subcore_axis_name="s"),
    compiler_params=pltpu.CompilerParams(
        needs_layout_passes=False, disable_bounds_checks=True),
    scratch_types=[
        pltpu.VMEM((16,), jnp.int32),
        pltpu.VMEM((16,), jnp.int32),
        pltpu.VMEM((64,), jnp.int32),
        pltpu.VMEM((1, 128), jnp.float32),
        pltpu.VMEM((1, 128), jnp.float32),
        pltpu.VMEM((1, F_DIM, 128), jnp.float32),
        pltpu.VMEM((1, F_DIM, 128), jnp.float32),
        pltpu.VMEM((NSLOT, F_DIM, 128), jnp.float32),
        pltpu.VMEM((16,), jnp.float32),
        pltpu.SemaphoreType.DMA,
        pltpu.SemaphoreType.DMA,
        pltpu.SemaphoreType.DMA,
        pltpu.SemaphoreType.DMA,
    ],
)(_svdpp_body)


def kernel(user, item, implicit_items, user_biases, item_biases, P, Q, Y):
    return _svdpp(
        user.astype(jnp.int32),
        item.astype(jnp.int32),
        implicit_items.astype(jnp.int32),
        user_biases.T,
        item_biases.T,
        P.T,
        Q.T,
        Y.T,
    )


# speedup vs baseline: 5.6475x; 1.6155x over previous
import functools

import jax
import jax.numpy as jnp
from jax import lax
from jax.experimental import pallas as pl
from jax.experimental.pallas import tpu as pltpu
from jax.experimental.pallas import tpu_sc as plsc

VOCAB = 100000
F_DIM = 64
HIST = 50
MU = 3.5
NORM = float(HIST) ** (-0.5)
NCHUNK = F_DIM // 16
NTILE = 16
KMAX = (HIST + NTILE - 1) // NTILE


def _base_lane(idx):
    return pl.multiple_of((idx >> 7) << 7, 128), idx & 127


def _svdpp_body(user_hbm, item_hbm, imp_hbm, ub_hbm, ib_hbm, P_hbm, Q_hbm,
                Y_hbm, out_hbm, user_v, item_v, imp_v, bu_v, bi_v, pu_v,
                qi_v, rows_v, part_v, bvec_v, fin_y, fin_pq, fin_b, res_v,
                sh_y, sh_pq, sh_b, sem0, sem1, sem2, sem3):
    cid = lax.axis_index("c")
    sid = lax.axis_index("s")

    @pl.when(cid == 0)
    def _():
        iota = lax.iota(jnp.int32, 16)

        def col_chunks(ref3, slot, lane):
            slots = jnp.full((16,), slot, jnp.int32)
            lanes = jnp.full((16,), lane, jnp.int32)
            return [
                plsc.load_gather(ref3, [slots, iota + (c * 16), lanes])
                for c in range(NCHUNK)
            ]

        c0 = pltpu.async_copy(user_hbm, user_v.at[pl.ds(0, 1)], sem0)
        c1 = pltpu.async_copy(item_hbm, item_v.at[pl.ds(0, 1)], sem1)
        c2 = pltpu.async_copy(imp_hbm, imp_v.at[pl.ds(0, HIST)], sem2)
        c2.wait()

        valids, lanes_k = [], []
        ycopies = []
        for k in range(KMAX):
            j = sid + (k * NTILE)
            valid = j < HIST
            ivec = plsc.load_gather(
                imp_v, [jnp.full((16,), jnp.where(valid, j, 0), jnp.int32)])
            idx = jnp.where(valid, ivec[0], 0)
            base, lane = _base_lane(idx)
            valids.append(valid)
            lanes_k.append(lane)
            ycopies.append(pltpu.async_copy(
                Y_hbm.at[:, pl.ds(base, 128)], rows_v.at[k], sem3))

        c0.wait()
        c1.wait()

        @pl.when(sid == 2)
        def _():
            u = user_v[...][0]
            base, lane = _base_lane(u)
            gP = pltpu.async_copy(P_hbm.at[:, pl.ds(base, 128)],
                                  pu_v.at[0], sem0)
            gb = pltpu.async_copy(ub_hbm.at[:, pl.ds(base, 128)], bu_v, sem1)
            gP.wait()
            gb.wait()
            pc = col_chunks(pu_v, 0, lane)
            for c in range(NCHUNK):
                part_v[pl.ds(F_DIM + c * 16, 16)] = pc[c]
            bvec_v[...] = plsc.load_gather(
                bu_v, [jnp.zeros((16,), jnp.int32),
                       jnp.full((16,), lane, jnp.int32)])
            pltpu.sync_copy(part_v.at[pl.ds(F_DIM, F_DIM)], sh_pq.at[0])
            pltpu.sync_copy(bvec_v, sh_b.at[0])

        @pl.when(sid == 3)
        def _():
            it = item_v[...][0]
            base, lane = _base_lane(it)
            gQ = pltpu.async_copy(Q_hbm.at[:, pl.ds(base, 128)],
                                  qi_v.at[0], sem0)
            gb = pltpu.async_copy(ib_hbm.at[:, pl.ds(base, 128)], bi_v, sem1)
            gQ.wait()
            gb.wait()
            qc = col_chunks(qi_v, 0, lane)
            for c in range(NCHUNK):
                part_v[pl.ds(F_DIM + c * 16, 16)] = qc[c]
            bvec_v[...] = plsc.load_gather(
                bi_v, [jnp.zeros((16,), jnp.int32),
                       jnp.full((16,), lane, jnp.int32)])
            pltpu.sync_copy(part_v.at[pl.ds(F_DIM, F_DIM)], sh_pq.at[1])
            pltpu.sync_copy(bvec_v, sh_b.at[1])

        zero = jnp.zeros((16,), jnp.float32)
        acc = [zero] * NCHUNK
        for k in range(KMAX):
            ycopies[k].wait()
            row = col_chunks(rows_v, k, lanes_k[k])
            for c in range(NCHUNK):
                acc[c] = acc[c] + jnp.where(valids[k], row[c], zero)
        for c in range(NCHUNK):
            part_v[pl.ds(c * 16, 16)] = acc[c]
        pltpu.sync_copy(part_v.at[pl.ds(0, F_DIM)], sh_y.at[sid])

        plsc.subcore_barrier()

        @pl.when(sid == 0)
        def _():
            pltpu.sync_copy(sh_y, fin_y)
            pltpu.sync_copy(sh_pq, fin_pq)
            pltpu.sync_copy(sh_b, fin_b)
            total = None
            for c in range(NCHUNK):
                sl = slice(c * 16, (c + 1) * 16)
                a = fin_y[0, sl]
                for w in range(1, NTILE):
                    a = a + fin_y[w, sl]
                t = fin_pq[0, sl] * (fin_pq[1, sl] + NORM * a)
                total = t if total is None else total + t
            s = total[0]
            for i in range(1, 16):
                s = s + total[i]
            bu = fin_b[0, :][0]
            bi = fin_b[1, :][0]
            r = MU + bu + bi + s
            res_v[...] = jnp.full((16,), r, jnp.float32)
            pltpu.sync_copy(res_v.at[pl.ds(0, 1)], out_hbm)


_svdpp = functools.partial(
    pl.kernel,
    out_type=jax.ShapeDtypeStruct((1,), jnp.float32),
    mesh=plsc.VectorSubcoreMesh(core_axis_name="c", subcore_axis_name="s"),
    compiler_params=pltpu.CompilerParams(
        needs_layout_passes=False, disable_bounds_checks=True),
    scratch_types=[
        pltpu.VMEM((16,), jnp.int32),
        pltpu.VMEM((16,), jnp.int32),
        pltpu.VMEM((64,), jnp.int32),
        pltpu.VMEM((1, 128), jnp.float32),
        pltpu.VMEM((1, 128), jnp.float32),
        pltpu.VMEM((1, F_DIM, 128), jnp.float32),
        pltpu.VMEM((1, F_DIM, 128), jnp.float32),
        pltpu.VMEM((KMAX, F_DIM, 128), jnp.float32),
        pltpu.VMEM((2 * F_DIM,), jnp.float32),
        pltpu.VMEM((16,), jnp.float32),
        pltpu.VMEM((NTILE, F_DIM), jnp.float32),
        pltpu.VMEM((2, F_DIM), jnp.float32),
        pltpu.VMEM((2, 16), jnp.float32),
        pltpu.VMEM((16,), jnp.float32),
        pltpu.VMEM_SHARED((NTILE, F_DIM), jnp.float32),
        pltpu.VMEM_SHARED((2, F_DIM), jnp.float32),
        pltpu.VMEM_SHARED((2, 16), jnp.float32),
        pltpu.SemaphoreType.DMA,
        pltpu.SemaphoreType.DMA,
        pltpu.SemaphoreType.DMA,
        pltpu.SemaphoreType.DMA,
    ],
)(_svdpp_body)


def kernel(user, item, implicit_items, user_biases, item_biases, P, Q, Y):
    return _svdpp(
        user.astype(jnp.int32),
        item.astype(jnp.int32),
        implicit_items.astype(jnp.int32),
        user_biases.T,
        item_biases.T,
        P.T,
        Q.T,
        Y.T,
    )
